# Initial kernel scaffold; baseline (speedup 1.0000x reference)
import jax
import jax.numpy as jnp
from jax.experimental import pallas as pl


def kernel(points, features, params):
    raise NotImplementedError("write your pallas kernel here")


# speedup vs baseline: 4.3757x; 4.3757x over previous
import functools

import jax
import jax.numpy as jnp
from jax.experimental import pallas as pl
from jax.experimental.pallas import tpu as pltpu

_B = 128
_N = 256
_K = 7
_EPS = 1e-5

_HIGH = jax.lax.Precision.HIGHEST
_DEF = jax.lax.Precision.DEFAULT


def _knn_onehots(coords):
    inner = jax.lax.dot_general(coords, coords, (((1,), (1,)), ((), ())),
                                precision=_HIGH,
                                preferred_element_type=jnp.float32)
    sq = jnp.sum(coords * coords, axis=1)
    dist = sq[:, None] + sq[None, :] - 2.0 * inner
    iota = jax.lax.broadcasted_iota(jnp.int32, (_N, _N), 1)
    onehots = []
    for j in range(_K + 1):
        mv = jnp.min(dist, axis=1, keepdims=True)
        cand = dist <= mv
        ii = jnp.where(cand, iota, _N)
        am = jnp.min(ii, axis=1, keepdims=True)
        oh = iota == am
        if j > 0:
            onehots.append(oh.astype(jnp.float32))
        if j < _K:
            dist = jnp.where(oh, jnp.inf, dist)
    return jnp.concatenate(onehots, axis=0)


def _edge_conv(coords, fts, w):
    O = _knn_onehots(coords)
    nb = jnp.dot(O, fts, precision=_HIGH,
                 preferred_element_type=jnp.float32)
    base = jnp.dot(fts, w['wa'], precision=_HIGH,
                   preferred_element_type=jnp.float32) + w['b1']
    y = jnp.dot(nb, w['wb'], precision=_HIGH,
                preferred_element_type=jnp.float32)
    y = jax.nn.relu(y + jnp.tile(base, (_K, 1)))
    y = jax.nn.relu(jnp.dot(y, w['w2'], precision=_HIGH,
                            preferred_element_type=jnp.float32) + w['b2'])
    y = jax.nn.relu(jnp.dot(y, w['w3'], precision=_HIGH,
                            preferred_element_type=jnp.float32) + w['b3'])
    cout = y.shape[-1]
    agg = y[0:_N]
    for j in range(1, _K):
        agg = agg + y[j * _N:(j + 1) * _N]
    agg = agg * (1.0 / _K)
    sc = jnp.dot(fts, w['wsc'], precision=_HIGH,
                 preferred_element_type=jnp.float32) + w['bsc']
    return jax.nn.relu(agg + sc)


def _body(pts_ref, fts_ref, bn_s_ref, bn_b_ref,
          wa1, wb1, b11, w21, b21, w31, b31, wsc1, bsc1,
          wa2, wb2, b12, w22, b22, w32, b32, wsc2, bsc2,
          fc1w, fc1b, fc2w, fc2b,
          out_ref, pooled_scr):
    b = pl.program_id(0)
    pts = pts_ref[0]
    fts = fts_ref[0] * bn_s_ref[0] + bn_b_ref[0]

    blk1 = dict(wa=wa1[...], wb=wb1[...], b1=b11[...], w2=w21[...],
                b2=b21[...], w3=w31[...], b3=b31[...], wsc=wsc1[...],
                bsc=bsc1[...])
    blk2 = dict(wa=wa2[...], wb=wb2[...], b1=b12[...], w2=w22[...],
                b2=b22[...], w3=w32[...], b3=b32[...], wsc=wsc2[...],
                bsc=bsc2[...])

    fts1 = _edge_conv(pts, fts, blk1)
    fts2 = _edge_conv(fts1, fts1, blk2)
    pooled = jnp.mean(fts2, axis=0, keepdims=True)
    pooled_scr[pl.ds(b, 1), :] = pooled

    @pl.when(b == _B - 1)
    def _head():
        p = pooled_scr[...]
        h = jnp.dot(p, fc1w[...], precision=_HIGH,
                    preferred_element_type=jnp.float32) + fc1b[...]
        h = h * jnp.tanh(jax.nn.softplus(h))
        out_ref[...] = jnp.dot(h, fc2w[...], precision=_HIGH,
                               preferred_element_type=jnp.float32) + fc2b[...]


def _fold_block(p):
    out = {}
    w0 = p['conv_w'][0]
    c = w0.shape[1] // 2
    s0 = p['bn_g'][0] / jnp.sqrt(1.0 + _EPS)
    w1, w2 = w0[:, :c], w0[:, c:]
    out['wa'] = (s0[:, None] * (w1 - w2)).T
    out['wb'] = (s0[:, None] * w2).T
    out['b1'] = p['bn_b'][0][None, :]
    s1 = p['bn_g'][1] / jnp.sqrt(1.0 + _EPS)
    out['w2'] = (s1[:, None] * p['conv_w'][1]).T
    out['b2'] = p['bn_b'][1][None, :]
    s2 = p['bn_g'][2] / jnp.sqrt(1.0 + _EPS)
    out['w3'] = (s2[:, None] * p['conv_w'][2]).T
    out['b3'] = p['bn_b'][2][None, :]
    ssc = p['sc_g'] / jnp.sqrt(1.0 + _EPS)
    out['wsc'] = (ssc[:, None] * p['sc_w']).T
    out['bsc'] = p['sc_b'][None, :]
    return out


@functools.partial(jax.jit, static_argnames=())
def kernel(points, features, params):
    pts_t = jnp.transpose(points[:, 0], (0, 2, 1))
    fts_t = jnp.transpose(features[:, 0], (0, 2, 1))
    bn_s = (params['bn_fts_g'] / jnp.sqrt(1.0 + _EPS))[None, :]
    bn_b = params['bn_fts_b'][None, :]
    f1 = _fold_block(params['blocks'][0])
    f2 = _fold_block(params['blocks'][1])
    fc1w = params['fc1_w'].T
    fc1b = params['fc1_b'][None, :]
    fc2w = params['fc2_w'].T
    fc2b = params['fc2_b'][None, :]

    def cspec(shape):
        nd = len(shape)
        return pl.BlockSpec(shape, lambda b: (0,) * nd)

    in_specs = [
        pl.BlockSpec((1, _N, 2), lambda b: (b, 0, 0)),
        pl.BlockSpec((1, _N, 16), lambda b: (b, 0, 0)),
        cspec(bn_s.shape), cspec(bn_b.shape),
    ]
    weight_ops = [f1['wa'], f1['wb'], f1['b1'], f1['w2'], f1['b2'],
                  f1['w3'], f1['b3'], f1['wsc'], f1['bsc'],
                  f2['wa'], f2['wb'], f2['b1'], f2['w2'], f2['b2'],
                  f2['w3'], f2['b3'], f2['wsc'], f2['bsc'],
                  fc1w, fc1b, fc2w, fc2b]
    in_specs += [cspec(w.shape) for w in weight_ops]

    out = pl.pallas_call(
        _body,
        grid=(_B,),
        in_specs=in_specs,
        out_specs=pl.BlockSpec((_B, 2), lambda b: (0, 0)),
        out_shape=jax.ShapeDtypeStruct((_B, 2), jnp.float32),
        scratch_shapes=[pltpu.VMEM((_B, 64), jnp.float32)],
        compiler_params=pltpu.CompilerParams(
            dimension_semantics=("arbitrary",),
        ),
    )(pts_t, fts_t, bn_s, bn_b, *weight_ops)
    return out


# speedup vs baseline: 14.5053x; 3.3150x over previous
import jax
import jax.numpy as jnp
from jax.experimental import pallas as pl
from jax.experimental.pallas import tpu as pltpu

_B = 128
_N = 256
_K = 7
_E = 4
_EPS = 1e-5

_HIGHEST = jax.lax.Precision.HIGHEST


def _split(x):
    hi = x.astype(jnp.bfloat16)
    lo = (x - hi.astype(jnp.float32)).astype(jnp.bfloat16)
    return hi, lo


def _mm3(a, b, dnums=None):
    ah, al = _split(a)
    bh, bl = _split(b)
    if dnums is None:
        dnums = (((a.ndim - 1,), (0,)), ((), ()))
    dot = lambda x, y: jax.lax.dot_general(
        x, y, dnums, preferred_element_type=jnp.float32)
    return dot(ah, bh) + (dot(ah, bl) + dot(al, bh))


def _mm2(a_bf16, b):
    bh, bl = _split(b)
    return (jnp.dot(a_bf16, bh, preferred_element_type=jnp.float32)
            + jnp.dot(a_bf16, bl, preferred_element_type=jnp.float32))


def _selectors(coords):
    inner = _mm3(coords, coords, (((1,), (1,)), ((), ())))
    sq = jnp.sum(coords * coords, axis=1)
    dist = sq[:, None] + sq[None, :] - 2.0 * inner
    sels = []
    for j in range(_K + 1):
        mv = jnp.min(dist, axis=1, keepdims=True)
        cand = dist <= mv
        if j > 0:
            sels.append(cand)
        if j < _K:
            dist = jnp.where(cand, jnp.inf, dist)
    return jnp.concatenate(sels, axis=0).astype(jnp.bfloat16)


def _edge_conv(coords, fts, w):
    O = _selectors(coords)
    G = jnp.dot(fts, w['wb'], precision=_HIGHEST,
                preferred_element_type=jnp.float32)
    base = jnp.dot(fts, w['wa'], precision=_HIGHEST,
                   preferred_element_type=jnp.float32) + w['b1']
    y = _mm2(O, G)
    y = jax.nn.relu(y + jnp.tile(base, (_K, 1)))
    y = jax.nn.relu(_mm3(y, w['w2']) + w['b2'])
    y = jax.nn.relu(_mm3(y, w['w3']) + w['b3'])
    agg = y[0:_N]
    for j in range(1, _K):
        agg = agg + y[j * _N:(j + 1) * _N]
    agg = agg * (1.0 / _K)
    sc = jnp.dot(fts, w['wsc'], precision=_HIGHEST,
                 preferred_element_type=jnp.float32) + w['bsc']
    return jax.nn.relu(agg + sc)


def _body(pts_ref, fts_ref, bn_s_ref, bn_b_ref,
          wa1, wb1, b11, w21, b21, w31, b31, wsc1, bsc1,
          wa2, wb2, b12, w22, b22, w32, b32, wsc2, bsc2,
          fc1w, fc1b, fc2w, fc2b,
          out_ref, pooled_scr):
    b = pl.program_id(0)

    blk1 = dict(wa=wa1[...], wb=wb1[...], b1=b11[...], w2=w21[...],
                b2=b21[...], w3=w31[...], b3=b31[...], wsc=wsc1[...],
                bsc=bsc1[...])
    blk2 = dict(wa=wa2[...], wb=wb2[...], b1=b12[...], w2=w22[...],
                b2=b22[...], w3=w32[...], b3=b32[...], wsc=wsc2[...],
                bsc=bsc2[...])

    for e in range(_E):
        pts = pts_ref[e]
        fts = fts_ref[e] * bn_s_ref[0] + bn_b_ref[0]
        fts1 = _edge_conv(pts, fts, blk1)
        fts2 = _edge_conv(fts1, fts1, blk2)
        pooled = jnp.mean(fts2, axis=0, keepdims=True)
        pooled_scr[pl.ds(b * _E + e, 1), :] = pooled

    @pl.when(b == _B // _E - 1)
    def _head():
        p = pooled_scr[...]
        h = jnp.dot(p, fc1w[...], precision=_HIGHEST,
                    preferred_element_type=jnp.float32) + fc1b[...]
        h = h * jnp.tanh(jax.nn.softplus(h))
        out_ref[...] = jnp.dot(h, fc2w[...], precision=_HIGHEST,
                               preferred_element_type=jnp.float32) + fc2b[...]


def _fold_block(p):
    out = {}
    w0 = p['conv_w'][0]
    c = w0.shape[1] // 2
    s0 = p['bn_g'][0] / jnp.sqrt(1.0 + _EPS)
    w1, w2 = w0[:, :c], w0[:, c:]
    out['wa'] = (s0[:, None] * (w1 - w2)).T
    out['wb'] = (s0[:, None] * w2).T
    out['b1'] = p['bn_b'][0][None, :]
    s1 = p['bn_g'][1] / jnp.sqrt(1.0 + _EPS)
    out['w2'] = (s1[:, None] * p['conv_w'][1]).T
    out['b2'] = p['bn_b'][1][None, :]
    s2 = p['bn_g'][2] / jnp.sqrt(1.0 + _EPS)
    out['w3'] = (s2[:, None] * p['conv_w'][2]).T
    out['b3'] = p['bn_b'][2][None, :]
    ssc = p['sc_g'] / jnp.sqrt(1.0 + _EPS)
    out['wsc'] = (ssc[:, None] * p['sc_w']).T
    out['bsc'] = p['sc_b'][None, :]
    return out


def kernel(points, features, params):
    pts_t = jnp.transpose(points[:, 0], (0, 2, 1))
    fts_t = jnp.transpose(features[:, 0], (0, 2, 1))
    bn_s = (params['bn_fts_g'] / jnp.sqrt(1.0 + _EPS))[None, :]
    bn_b = params['bn_fts_b'][None, :]
    f1 = _fold_block(params['blocks'][0])
    f2 = _fold_block(params['blocks'][1])
    fc1w = params['fc1_w'].T
    fc1b = params['fc1_b'][None, :]
    fc2w = params['fc2_w'].T
    fc2b = params['fc2_b'][None, :]

    def cspec(shape):
        nd = len(shape)
        return pl.BlockSpec(shape, lambda b: (0,) * nd)

    in_specs = [
        pl.BlockSpec((_E, _N, 2), lambda b: (b, 0, 0)),
        pl.BlockSpec((_E, _N, 16), lambda b: (b, 0, 0)),
        cspec(bn_s.shape), cspec(bn_b.shape),
    ]
    weight_ops = [f1['wa'], f1['wb'], f1['b1'], f1['w2'], f1['b2'],
                  f1['w3'], f1['b3'], f1['wsc'], f1['bsc'],
                  f2['wa'], f2['wb'], f2['b1'], f2['w2'], f2['b2'],
                  f2['w3'], f2['b3'], f2['wsc'], f2['bsc'],
                  fc1w, fc1b, fc2w, fc2b]
    in_specs += [cspec(w.shape) for w in weight_ops]

    out = pl.pallas_call(
        _body,
        grid=(_B // _E,),
        in_specs=in_specs,
        out_specs=pl.BlockSpec((_B, 2), lambda b: (0, 0)),
        out_shape=jax.ShapeDtypeStruct((_B, 2), jnp.float32),
        scratch_shapes=[pltpu.VMEM((_B, 64), jnp.float32)],
        compiler_params=pltpu.CompilerParams(
            dimension_semantics=("arbitrary",),
        ),
    )(pts_t, fts_t, bn_s, bn_b, *weight_ops)
    return out
